# baseline (speedup 1.0000x reference)
import functools

import jax
import jax.numpy as jnp
from jax import lax
from jax.experimental import pallas as pl
from jax.experimental.pallas import tpu as pltpu
from jax.experimental.pallas import tpu_sc as plsc

N = 10000
E = 320000
ET = E + N
R = 10240
NW = 32
NBUF = 3
NI = 6
RPT = R // 16
BR = 256


def _prep_body(heads, oc, x_ref, w_ref, asr_ref, adr_ref, tab_ref, sdst_ref):
    xb = x_ref[...]
    h = jnp.dot(xb, w_ref[...], preferred_element_type=jnp.float32)
    ones = jnp.ones((BR, 1), jnp.float32)
    zpad = jnp.zeros((BR, 14), jnp.float32)
    for k in range(heads):
        hk = h[:, k * oc:(k + 1) * oc]
        ssrc = jnp.sum(hk * asr_ref[k][None, :], axis=1, keepdims=True)
        tab_ref[k] = jnp.concatenate([hk, ones, ssrc, zpad], axis=1)
        sdst_ref[k] = jnp.sum(hk * adr_ref[k][None, :], axis=1)


def _prep(x, W, a_src, a_dst, heads, oc):
    ind = x.shape[1]
    C = oc + 16
    grid = R // BR
    return pl.pallas_call(
        functools.partial(_prep_body, heads, oc),
        grid=(grid,),
        in_specs=[
            pl.BlockSpec((BR, ind), lambda i: (i, 0)),
            pl.BlockSpec((ind, heads * oc), lambda i: (0, 0)),
            pl.BlockSpec((heads, oc), lambda i: (0, 0)),
            pl.BlockSpec((heads, oc), lambda i: (0, 0)),
        ],
        out_specs=[
            pl.BlockSpec((heads, BR, C), lambda i: (0, i, 0)),
            pl.BlockSpec((heads, BR), lambda i: (0, i)),
        ],
        out_shape=[
            jax.ShapeDtypeStruct((heads, R, C), jnp.float32),
            jax.ShapeDtypeStruct((heads, R), jnp.float32),
        ],
    )(x, W, a_src, a_dst)


def _fin_body(heads, oc, relu, bias_ref, acc_ref, o_ref):
    acc = acc_ref[...]
    num = acc[0, :, :, :oc] + acc[1, :, :, :oc]
    z = acc[0, :, :, oc:oc + 1] + acc[1, :, :, oc:oc + 1]
    o = jnp.sum(num / (z + 1e-16), axis=0) * (1.0 / heads) + bias_ref[...][None, :]
    if relu:
        o = jnp.maximum(o, 0.0)
    o_ref[...] = o


def _fin(acc, bias, heads, oc, relu):
    C = oc + 16
    grid = R // BR
    return pl.pallas_call(
        functools.partial(_fin_body, heads, oc, relu),
        grid=(grid,),
        in_specs=[
            pl.BlockSpec((oc,), lambda i: (0,)),
            pl.BlockSpec((2, heads, BR, C), lambda i: (0, 0, i, 0)),
        ],
        out_specs=pl.BlockSpec((BR, oc), lambda i: (i, 0)),
        out_shape=jax.ShapeDtypeStruct((R, oc), jnp.float32),
    )(bias, acc)


def _pool_body(h_ref, b_ref, wg1_ref, bg1_ref, wg2_ref, bg2_ref, o_ref):
    hh = h_ref[...]
    t = jnp.maximum(
        jnp.dot(hh, wg1_ref[...], preferred_element_type=jnp.float32)
        + bg1_ref[...][None, :], 0.0)
    g = jnp.dot(t, wg2_ref[...], preferred_element_type=jnp.float32) \
        + bg2_ref[...]
    oh = (b_ref[...] == lax.broadcasted_iota(jnp.int32, (R, 16), 1)).astype(
        jnp.float32)
    m = jnp.max(jnp.where(oh > 0, g, -3e38), axis=0, keepdims=True)
    mn = jnp.sum(oh * m, axis=1, keepdims=True)
    p = jnp.exp(g - mn)
    z = jnp.sum(oh * p, axis=0, keepdims=True)
    zn = jnp.sum(oh * z, axis=1, keepdims=True)
    w = oh * (p / (zn + 1e-16))
    pooled = lax.dot_general(w, hh, (((0,), (0,)), ((), ())),
                             preferred_element_type=jnp.float32)
    o_ref[...] = jnp.dot(oh, pooled, preferred_element_type=jnp.float32)


def _pool(h, batch2d, Wg1, bg1, Wg2, bg2):
    return pl.pallas_call(
        _pool_body,
        out_shape=jax.ShapeDtypeStruct((R, 64), jnp.float32),
    )(h, batch2d, Wg1, bg1, Wg2, bg2)


def _edge_body(heads, C, BB, NB, tab, srcdst, sdst, acc_out, rb, cmb, sdst_t,
               pb, acc_sh, *sems):
    gsems = sems[0:NBUF]
    ssems = sems[NBUF:2 * NBUF]
    isems = sems[2 * NBUF:2 * NBUF + NI]
    c = lax.axis_index("c")
    s = lax.axis_index("s")
    wid = s * 2 + c
    nj = C // 16
    iota16 = lax.broadcasted_iota(jnp.int32, (16,), 0)

    for h in range(heads):
        pltpu.sync_copy(sdst.at[h], sdst_t)

        def _zb(ei, _):
            for j in range(nj):
                rb[0, ei, pl.ds(j * 16, 16)] = jnp.zeros((16,), jnp.float32)
            return 0
        lax.fori_loop(0, BB, _zb, 0)
        for k in range(RPT // BB):
            pltpu.sync_copy(rb.at[0],
                            acc_sh.at[pl.ds(s * RPT + k * BB, BB)])
        rem = RPT - (RPT // BB) * BB
        if rem:
            pltpu.sync_copy(
                rb.at[0, pl.ds(0, rem)],
                acc_sh.at[pl.ds(s * RPT + (RPT // BB) * BB, rem)])
        plsc.subcore_barrier()

        for b0 in range(5):
            pltpu.async_copy(srcdst.at[h, wid, b0], cmb.at[b0], isems[b0])
        for b0 in range(2):
            pltpu.make_async_copy(srcdst.at[h, wid, b0], cmb.at[b0],
                                  isems[b0]).wait()
            pltpu.async_copy(tab.at[cmb.at[b0, 0]], rb.at[b0], gsems[b0])

        def _batch(b, par, q):
            nxt = (par + 2) % NBUF
            qn = (q + 2) % NI
            qf = (q + 5) % NI
            pltpu.make_async_copy(tab.at[cmb.at[0, 0]], rb.at[par],
                                  gsems[par]).wait()
            def _pw(g, _):
                g16 = g * 16
                d16 = cmb[q, 1, pl.ds(g16, 16)]
                ss = plsc.load_gather(
                    rb, [jnp.full((16,), par, jnp.int32),
                         g16 + iota16,
                         jnp.full((16,), C - 15, jnp.int32)])
                e = ss + plsc.load_gather(sdst_t, [d16])
                e = jnp.where(e >= 0, e, e * jnp.float32(0.2))
                pb[pl.ds(g16, 16)] = jnp.exp(e)
                return 0
            lax.fori_loop(0, BB // 16, _pw, 0)

            def _scale(ei, _):
                pe = jnp.full((16,), pb[pl.ds(ei, 16)][0], jnp.float32)
                for j in range(nj):
                    rb[par, ei, pl.ds(j * 16, 16)] = (
                        rb[par, ei, pl.ds(j * 16, 16)] * pe)
                return 0
            lax.fori_loop(0, BB, _scale, 0)

            pltpu.async_copy(rb.at[par], acc_sh.at[cmb.at[q, 1]],
                             ssems[par], add=True)

            @pl.when(b + 2 < NB)
            def _():
                @pl.when(b >= 1)
                def _():
                    pltpu.make_async_copy(rb.at[nxt],
                                          acc_sh.at[cmb.at[0, 1]],
                                          ssems[nxt]).wait()
                pltpu.make_async_copy(srcdst.at[h, wid, 0], cmb.at[qn],
                                      isems[qn]).wait()
                pltpu.async_copy(tab.at[cmb.at[qn, 0]], rb.at[nxt],
                                 gsems[nxt])

            @pl.when(b + 5 < NB)
            def _():
                pltpu.async_copy(srcdst.at[h, wid, b + 5], cmb.at[qf],
                                 isems[qf])

        def _hex(t, _):
            for u in range(NI):
                _batch(t * NI + u, u % NBUF, u)
            return 0
        lax.fori_loop(0, NB // NI, _hex, 0)
        for par in range(NBUF):
            pltpu.make_async_copy(rb.at[par], acc_sh.at[cmb.at[0, 1]],
                                  ssems[par]).wait()
        plsc.subcore_barrier()
        pltpu.sync_copy(acc_sh.at[pl.ds(s * RPT, RPT)],
                        acc_out.at[c, h, pl.ds(s * RPT, RPT)])
        plsc.subcore_barrier()


def _edge(tab, srcdst, sdst, heads, C, BB, NB):
    mesh = plsc.VectorSubcoreMesh(core_axis_name="c", subcore_axis_name="s")
    return pl.kernel(
        functools.partial(_edge_body, heads, C, BB, NB),
        out_type=jax.ShapeDtypeStruct((2, heads, R, C), jnp.float32),
        mesh=mesh,
        compiler_params=pltpu.CompilerParams(needs_layout_passes=False,
                                             use_tc_tiling_on_sc=False),
        scratch_types=[
            pltpu.VMEM((NBUF, BB, C), jnp.float32),
            pltpu.VMEM((NI, 2, BB), jnp.int32),
            pltpu.VMEM((R,), jnp.float32),
            pltpu.VMEM((BB + 16,), jnp.float32),
            pltpu.VMEM_SHARED((R, C), jnp.float32),
        ] + [pltpu.SemaphoreType.DMA] * (2 * NBUF + NI),
    )(tab, srcdst, sdst)


def kernel(x, edge_index, batch, W_e0, a_src_e0, a_dst_e0, b_e0, W_e1,
           a_src_e1, a_dst_e1, b_e1, Wg1, bg1, Wg2, bg2, W_d0, a_src_d0,
           a_dst_d0, b_d0, W_d1, a_src_d1, a_dst_d1, b_d1):
    f32 = jnp.float32
    i32 = jnp.int32

    x_pad = jnp.zeros((R, 128), f32).at[:N].set(x)
    loops = jnp.arange(N, dtype=i32)
    src0 = jnp.concatenate([edge_index[0].astype(i32), loops])
    dst0 = jnp.concatenate([edge_index[1].astype(i32), loops])
    batch2d = jnp.full((R,), 16, i32).at[:N].set(batch.astype(i32)).reshape(
        R, 1)

    def mk_srcdst(heads, BB, NB):
        EPL = NW * NB * BB
        srcp = jnp.full((EPL,), N, i32).at[:ET].set(src0)
        dstp = jnp.full((EPL,), N, i32).at[:ET].set(dst0)
        dst5 = jnp.broadcast_to(dstp.reshape(1, NW, NB, 1, BB),
                                (heads, NW, NB, 1, BB))
        srcoff = (srcp[None, :]
                  + (jnp.arange(heads, dtype=i32) * R)[:, None]
                  ).reshape(heads, NW, NB, 1, BB)
        return jnp.concatenate([srcoff, dst5], axis=3)

    sd_e0 = mk_srcdst(8, 64, 162)
    sd_e1 = mk_srcdst(8, 256, 42)
    sd_d = mk_srcdst(1, 64, 162)

    tab, sdst = _prep(x_pad, W_e0, a_src_e0.reshape(8, 128),
                      a_dst_e0.reshape(8, 128), 8, 128)
    acc = _edge(tab.reshape(8 * R, 144), sd_e0, sdst, 8, 144, 64, 162)
    h = _fin(acc, b_e0, 8, 128, relu=True)

    tab, sdst = _prep(h, W_e1, a_src_e1.reshape(8, 64),
                      a_dst_e1.reshape(8, 64), 8, 64)
    acc = _edge(tab.reshape(8 * R, 80), sd_e1, sdst, 8, 80, 256, 42)
    h = _fin(acc, b_e1, 8, 64, relu=False)

    h = _pool(h, batch2d, Wg1, bg1, Wg2, bg2.reshape(1, 1))

    tab, sdst = _prep(h, W_d0, a_src_d0.reshape(1, 128),
                      a_dst_d0.reshape(1, 128), 1, 128)
    acc = _edge(tab.reshape(R, 144), sd_d, sdst, 1, 144, 64, 162)
    h = _fin(acc, b_d0, 1, 128, relu=True)

    tab, sdst = _prep(h, W_d1, a_src_d1.reshape(1, 128),
                      a_dst_d1.reshape(1, 128), 1, 128)
    acc = _edge(tab.reshape(R, 144), sd_d, sdst, 1, 144, 64, 162)
    h = _fin(acc, b_d1, 1, 128, relu=False)

    return h[:N]


# speedup vs baseline: 1.0113x; 1.0113x over previous
import functools

import jax
import jax.numpy as jnp
from jax import lax
from jax.experimental import pallas as pl
from jax.experimental.pallas import tpu as pltpu
from jax.experimental.pallas import tpu_sc as plsc

N = 10000
E = 320000
ET = E + N
R = 10240
NW = 32
NBUF = 3
NI = 6
RPT = R // 16
BR = 256


def _prep_body(heads, oc, x_ref, w_ref, asr_ref, adr_ref, tab_ref, sdst_ref):
    xb = x_ref[...]
    h = jnp.dot(xb, w_ref[...], preferred_element_type=jnp.float32)
    ones = jnp.ones((BR, 1), jnp.float32)
    zpad = jnp.zeros((BR, 14), jnp.float32)
    for k in range(heads):
        hk = h[:, k * oc:(k + 1) * oc]
        ssrc = jnp.sum(hk * asr_ref[k][None, :], axis=1, keepdims=True)
        tab_ref[k] = jnp.concatenate([hk, ones, ssrc, zpad], axis=1)
        sdst_ref[k] = jnp.sum(hk * adr_ref[k][None, :], axis=1)


def _prep(x, W, a_src, a_dst, heads, oc):
    ind = x.shape[1]
    C = oc + 16
    grid = R // BR
    return pl.pallas_call(
        functools.partial(_prep_body, heads, oc),
        grid=(grid,),
        in_specs=[
            pl.BlockSpec((BR, ind), lambda i: (i, 0)),
            pl.BlockSpec((ind, heads * oc), lambda i: (0, 0)),
            pl.BlockSpec((heads, oc), lambda i: (0, 0)),
            pl.BlockSpec((heads, oc), lambda i: (0, 0)),
        ],
        out_specs=[
            pl.BlockSpec((heads, BR, C), lambda i: (0, i, 0)),
            pl.BlockSpec((heads, BR), lambda i: (0, i)),
        ],
        out_shape=[
            jax.ShapeDtypeStruct((heads, R, C), jnp.float32),
            jax.ShapeDtypeStruct((heads, R), jnp.float32),
        ],
    )(x, W, a_src, a_dst)


def _fin_body(heads, oc, relu, bias_ref, acc_ref, o_ref):
    acc = acc_ref[...]
    num = acc[0, :, :, :oc] + acc[1, :, :, :oc]
    z = acc[0, :, :, oc:oc + 1] + acc[1, :, :, oc:oc + 1]
    o = jnp.sum(num / (z + 1e-16), axis=0) * (1.0 / heads) + bias_ref[...][None, :]
    if relu:
        o = jnp.maximum(o, 0.0)
    o_ref[...] = o


def _fin(acc, bias, heads, oc, relu):
    C = oc + 16
    grid = R // BR
    return pl.pallas_call(
        functools.partial(_fin_body, heads, oc, relu),
        grid=(grid,),
        in_specs=[
            pl.BlockSpec((oc,), lambda i: (0,)),
            pl.BlockSpec((2, heads, BR, C), lambda i: (0, 0, i, 0)),
        ],
        out_specs=pl.BlockSpec((BR, oc), lambda i: (i, 0)),
        out_shape=jax.ShapeDtypeStruct((R, oc), jnp.float32),
    )(bias, acc)


def _pool_body(h_ref, b_ref, wg1_ref, bg1_ref, wg2_ref, bg2_ref, o_ref):
    hh = h_ref[...]
    t = jnp.maximum(
        jnp.dot(hh, wg1_ref[...], preferred_element_type=jnp.float32)
        + bg1_ref[...][None, :], 0.0)
    g = jnp.dot(t, wg2_ref[...], preferred_element_type=jnp.float32) \
        + bg2_ref[...]
    oh = (b_ref[...] == lax.broadcasted_iota(jnp.int32, (R, 16), 1)).astype(
        jnp.float32)
    m = jnp.max(jnp.where(oh > 0, g, -3e38), axis=0, keepdims=True)
    mn = jnp.sum(oh * m, axis=1, keepdims=True)
    p = jnp.exp(g - mn)
    z = jnp.sum(oh * p, axis=0, keepdims=True)
    zn = jnp.sum(oh * z, axis=1, keepdims=True)
    w = oh * (p / (zn + 1e-16))
    pooled = lax.dot_general(w, hh, (((0,), (0,)), ((), ())),
                             preferred_element_type=jnp.float32)
    o_ref[...] = jnp.dot(oh, pooled, preferred_element_type=jnp.float32)


def _pool(h, batch2d, Wg1, bg1, Wg2, bg2):
    return pl.pallas_call(
        _pool_body,
        out_shape=jax.ShapeDtypeStruct((R, 64), jnp.float32),
    )(h, batch2d, Wg1, bg1, Wg2, bg2)


def _edge_body(heads, C, BB, NB, tab, srcdst, sdst, acc_out, rb, cmb, sdst_t,
               pb, acc_sh, *sems):
    gsems = sems[0:NBUF]
    ssems = sems[NBUF:2 * NBUF]
    isems = sems[2 * NBUF:2 * NBUF + NI]
    c = lax.axis_index("c")
    s = lax.axis_index("s")
    wid = s * 2 + c
    nj = C // 16
    iota16 = lax.broadcasted_iota(jnp.int32, (16,), 0)

    for h in range(heads):
        pltpu.sync_copy(sdst.at[h], sdst_t)

        def _zb(ei, _):
            for j in range(nj):
                rb[0, ei, pl.ds(j * 16, 16)] = jnp.zeros((16,), jnp.float32)
            return 0
        lax.fori_loop(0, BB, _zb, 0)
        for k in range(RPT // BB):
            pltpu.sync_copy(rb.at[0],
                            acc_sh.at[pl.ds(s * RPT + k * BB, BB)])
        rem = RPT - (RPT // BB) * BB
        if rem:
            pltpu.sync_copy(
                rb.at[0, pl.ds(0, rem)],
                acc_sh.at[pl.ds(s * RPT + (RPT // BB) * BB, rem)])
        plsc.subcore_barrier()

        for b0 in range(5):
            pltpu.async_copy(srcdst.at[h, wid, b0], cmb.at[b0], isems[b0])
        for b0 in range(2):
            pltpu.make_async_copy(srcdst.at[h, wid, b0], cmb.at[b0],
                                  isems[b0]).wait()
            pltpu.async_copy(tab.at[cmb.at[b0, 0]], rb.at[b0], gsems[b0])

        def _batch(b, par, q):
            nxt = (par + 2) % NBUF
            qn = (q + 2) % NI
            qf = (q + 5) % NI
            pltpu.make_async_copy(tab.at[cmb.at[0, 0]], rb.at[par],
                                  gsems[par]).wait()
            def _pw(g, _):
                g16 = g * 16
                d16 = cmb[q, 1, pl.ds(g16, 16)]
                ss = plsc.load_gather(
                    rb, [jnp.full((16,), par, jnp.int32),
                         g16 + iota16,
                         jnp.full((16,), C - 15, jnp.int32)])
                e = ss + plsc.load_gather(sdst_t, [d16])
                e = jnp.where(e >= 0, e, e * jnp.float32(0.2))
                pb[pl.ds(g16, 16)] = jnp.exp(e)
                return 0
            lax.fori_loop(0, BB // 16, _pw, 0)

            def _scale(ei, _):
                pe = jnp.full((16,), pb[pl.ds(ei, 16)][0], jnp.float32)
                for j in range(nj):
                    rb[par, ei, pl.ds(j * 16, 16)] = (
                        rb[par, ei, pl.ds(j * 16, 16)] * pe)
                return 0
            lax.fori_loop(0, BB, _scale, 0)

            pltpu.async_copy(rb.at[par], acc_sh.at[cmb.at[q, 1]],
                             ssems[par], add=True)

            @pl.when(b + 2 < NB)
            def _():
                @pl.when(b >= 1)
                def _():
                    pltpu.make_async_copy(rb.at[nxt],
                                          acc_sh.at[cmb.at[0, 1]],
                                          ssems[nxt]).wait()
                pltpu.make_async_copy(srcdst.at[h, wid, 0], cmb.at[qn],
                                      isems[qn]).wait()
                pltpu.async_copy(tab.at[cmb.at[qn, 0]], rb.at[nxt],
                                 gsems[nxt])

            @pl.when(b + 5 < NB)
            def _():
                pltpu.async_copy(srcdst.at[h, wid, b + 5], cmb.at[qf],
                                 isems[qf])

        def _hex(t, _):
            for u in range(NI):
                _batch(t * NI + u, u % NBUF, u)
            return 0
        lax.fori_loop(0, NB // NI, _hex, 0)
        for par in range(NBUF):
            pltpu.make_async_copy(rb.at[par], acc_sh.at[cmb.at[0, 1]],
                                  ssems[par]).wait()
        plsc.subcore_barrier()
        pltpu.sync_copy(acc_sh.at[pl.ds(s * RPT, RPT)],
                        acc_out.at[c, h, pl.ds(s * RPT, RPT)])
        plsc.subcore_barrier()


def _edge(tab, srcdst, sdst, heads, C, BB, NB):
    mesh = plsc.VectorSubcoreMesh(core_axis_name="c", subcore_axis_name="s")
    return pl.kernel(
        functools.partial(_edge_body, heads, C, BB, NB),
        out_type=jax.ShapeDtypeStruct((2, heads, R, C), jnp.float32),
        mesh=mesh,
        compiler_params=pltpu.CompilerParams(needs_layout_passes=False,
                                             use_tc_tiling_on_sc=False),
        scratch_types=[
            pltpu.VMEM((NBUF, BB, C), jnp.float32),
            pltpu.VMEM((NI, 2, BB), jnp.int32),
            pltpu.VMEM((R,), jnp.float32),
            pltpu.VMEM((BB + 16,), jnp.float32),
            pltpu.VMEM_SHARED((R, C), jnp.float32),
        ] + [pltpu.SemaphoreType.DMA] * (2 * NBUF + NI),
    )(tab, srcdst, sdst)


def kernel(x, edge_index, batch, W_e0, a_src_e0, a_dst_e0, b_e0, W_e1,
           a_src_e1, a_dst_e1, b_e1, Wg1, bg1, Wg2, bg2, W_d0, a_src_d0,
           a_dst_d0, b_d0, W_d1, a_src_d1, a_dst_d1, b_d1):
    f32 = jnp.float32
    i32 = jnp.int32

    x_pad = jnp.zeros((R, 128), f32).at[:N].set(x)
    loops = jnp.arange(N, dtype=i32)
    src0 = jnp.concatenate([edge_index[0].astype(i32), loops])
    dst0 = jnp.concatenate([edge_index[1].astype(i32), loops])
    batch2d = jnp.full((R,), 16, i32).at[:N].set(batch.astype(i32)).reshape(
        R, 1)

    def mk_srcdst(heads, BB, NB):
        EPL = NW * NB * BB
        srcp = jnp.full((EPL,), N, i32).at[:ET].set(src0)
        dstp = jnp.full((EPL,), N, i32).at[:ET].set(dst0)
        dst5 = jnp.broadcast_to(dstp.reshape(1, NW, NB, 1, BB),
                                (heads, NW, NB, 1, BB))
        srcoff = (srcp[None, :]
                  + (jnp.arange(heads, dtype=i32) * R)[:, None]
                  ).reshape(heads, NW, NB, 1, BB)
        return jnp.concatenate([srcoff, dst5], axis=3)

    sd_e0 = mk_srcdst(8, 64, 162)
    sd_e1 = mk_srcdst(8, 128, 84)
    sd_d = mk_srcdst(1, 64, 162)

    tab, sdst = _prep(x_pad, W_e0, a_src_e0.reshape(8, 128),
                      a_dst_e0.reshape(8, 128), 8, 128)
    acc = _edge(tab.reshape(8 * R, 144), sd_e0, sdst, 8, 144, 64, 162)
    h = _fin(acc, b_e0, 8, 128, relu=True)

    tab, sdst = _prep(h, W_e1, a_src_e1.reshape(8, 64),
                      a_dst_e1.reshape(8, 64), 8, 64)
    acc = _edge(tab.reshape(8 * R, 80), sd_e1, sdst, 8, 80, 128, 84)
    h = _fin(acc, b_e1, 8, 64, relu=False)

    h = _pool(h, batch2d, Wg1, bg1, Wg2, bg2.reshape(1, 1))

    tab, sdst = _prep(h, W_d0, a_src_d0.reshape(1, 128),
                      a_dst_d0.reshape(1, 128), 1, 128)
    acc = _edge(tab.reshape(R, 144), sd_d, sdst, 1, 144, 64, 162)
    h = _fin(acc, b_d0, 1, 128, relu=True)

    tab, sdst = _prep(h, W_d1, a_src_d1.reshape(1, 128),
                      a_dst_d1.reshape(1, 128), 1, 128)
    acc = _edge(tab.reshape(R, 144), sd_d, sdst, 1, 144, 64, 162)
    h = _fin(acc, b_d1, 1, 128, relu=False)

    return h[:N]


# speedup vs baseline: 1.3931x; 1.3775x over previous
import functools

import jax
import jax.numpy as jnp
from jax import lax
from jax.experimental import pallas as pl
from jax.experimental.pallas import tpu as pltpu
from jax.experimental.pallas import tpu_sc as plsc

N = 10000
E = 320000
ET = E + N
R = 10240
NW = 32
NBUF = 3
NI = 6
RPT = R // 16
BR = 256


def _prep_body(heads, oc, x_ref, w_ref, asr_ref, adr_ref, tab_ref, sdst_ref):
    xb = x_ref[...]
    h = jnp.dot(xb, w_ref[...], preferred_element_type=jnp.float32)
    ones = jnp.ones((BR, 1), jnp.float32)
    zpad = jnp.zeros((BR, 14), jnp.float32)
    for k in range(heads):
        hk = h[:, k * oc:(k + 1) * oc]
        ssrc = jnp.sum(hk * asr_ref[k][None, :], axis=1, keepdims=True)
        tab_ref[k] = jnp.concatenate([hk, ones, ssrc, zpad], axis=1)
        sdst_ref[k] = jnp.sum(hk * adr_ref[k][None, :], axis=1)


def _prep(x, W, a_src, a_dst, heads, oc):
    ind = x.shape[1]
    C = oc + 16
    grid = R // BR
    return pl.pallas_call(
        functools.partial(_prep_body, heads, oc),
        grid=(grid,),
        in_specs=[
            pl.BlockSpec((BR, ind), lambda i: (i, 0)),
            pl.BlockSpec((ind, heads * oc), lambda i: (0, 0)),
            pl.BlockSpec((heads, oc), lambda i: (0, 0)),
            pl.BlockSpec((heads, oc), lambda i: (0, 0)),
        ],
        out_specs=[
            pl.BlockSpec((heads, BR, C), lambda i: (0, i, 0)),
            pl.BlockSpec((heads, BR), lambda i: (0, i)),
        ],
        out_shape=[
            jax.ShapeDtypeStruct((heads, R, C), jnp.float32),
            jax.ShapeDtypeStruct((heads, R), jnp.float32),
        ],
    )(x, W, a_src, a_dst)


def _fin_body(heads, oc, relu, bias_ref, acc_ref, o_ref):
    acc = acc_ref[...]
    num = acc[0, :, :, :oc] + acc[1, :, :, :oc]
    z = acc[0, :, :, oc:oc + 1] + acc[1, :, :, oc:oc + 1]
    o = jnp.sum(num / (z + 1e-16), axis=0) * (1.0 / heads) + bias_ref[...][None, :]
    if relu:
        o = jnp.maximum(o, 0.0)
    o_ref[...] = o


def _fin(acc, bias, heads, oc, relu):
    C = oc + 16
    grid = R // BR
    return pl.pallas_call(
        functools.partial(_fin_body, heads, oc, relu),
        grid=(grid,),
        in_specs=[
            pl.BlockSpec((oc,), lambda i: (0,)),
            pl.BlockSpec((2, heads, BR, C), lambda i: (0, 0, i, 0)),
        ],
        out_specs=pl.BlockSpec((BR, oc), lambda i: (i, 0)),
        out_shape=jax.ShapeDtypeStruct((R, oc), jnp.float32),
    )(bias, acc)


def _pool_body(h_ref, b_ref, wg1_ref, bg1_ref, wg2_ref, bg2_ref, o_ref):
    hh = h_ref[...]
    t = jnp.maximum(
        jnp.dot(hh, wg1_ref[...], preferred_element_type=jnp.float32)
        + bg1_ref[...][None, :], 0.0)
    g = jnp.dot(t, wg2_ref[...], preferred_element_type=jnp.float32) \
        + bg2_ref[...]
    oh = (b_ref[...] == lax.broadcasted_iota(jnp.int32, (R, 16), 1)).astype(
        jnp.float32)
    m = jnp.max(jnp.where(oh > 0, g, -3e38), axis=0, keepdims=True)
    mn = jnp.sum(oh * m, axis=1, keepdims=True)
    p = jnp.exp(g - mn)
    z = jnp.sum(oh * p, axis=0, keepdims=True)
    zn = jnp.sum(oh * z, axis=1, keepdims=True)
    w = oh * (p / (zn + 1e-16))
    pooled = lax.dot_general(w, hh, (((0,), (0,)), ((), ())),
                             preferred_element_type=jnp.float32)
    o_ref[...] = jnp.dot(oh, pooled, preferred_element_type=jnp.float32)


def _pool(h, batch2d, Wg1, bg1, Wg2, bg2):
    return pl.pallas_call(
        _pool_body,
        out_shape=jax.ShapeDtypeStruct((R, 64), jnp.float32),
    )(h, batch2d, Wg1, bg1, Wg2, bg2)


def _edge_body(heads, C, BB, NB, tab, srcdst, sdst, acc_out, rb, cmb, sdst_t,
               pb, acc_sh, *sems):
    gsems = sems[0:NBUF]
    ssems = sems[NBUF:2 * NBUF]
    isems = sems[2 * NBUF:2 * NBUF + NI]
    c = lax.axis_index("c")
    s = lax.axis_index("s")
    wid = s * 2 + c
    nj = C // 16
    iota16 = lax.broadcasted_iota(jnp.int32, (16,), 0)

    for h in range(heads):
        pltpu.sync_copy(sdst.at[h], sdst_t)

        def _zb(ei, _):
            for j in range(nj):
                rb[0, ei, pl.ds(j * 16, 16)] = jnp.zeros((16,), jnp.float32)
            return 0
        lax.fori_loop(0, BB, _zb, 0)
        for k in range(RPT // BB):
            pltpu.sync_copy(rb.at[0],
                            acc_sh.at[pl.ds(s * RPT + k * BB, BB)])
        rem = RPT - (RPT // BB) * BB
        if rem:
            pltpu.sync_copy(
                rb.at[0, pl.ds(0, rem)],
                acc_sh.at[pl.ds(s * RPT + (RPT // BB) * BB, rem)])
        plsc.subcore_barrier()

        for b0 in range(5):
            pltpu.async_copy(srcdst.at[h, wid, b0], cmb.at[b0], isems[b0])
        for b0 in range(2):
            pltpu.make_async_copy(srcdst.at[h, wid, b0], cmb.at[b0],
                                  isems[b0]).wait()
            pltpu.async_copy(tab.at[cmb.at[b0, 0]], rb.at[b0], gsems[b0])

        def _batch(b, par, q):
            nxt = (par + 2) % NBUF
            qn = (q + 2) % NI
            qf = (q + 5) % NI
            pltpu.make_async_copy(tab.at[cmb.at[0, 0]], rb.at[par],
                                  gsems[par]).wait()
            def _pw(g, _):
                g16 = g * 16
                d16 = cmb[q, 1, pl.ds(g16, 16)]
                ss = plsc.load_gather(
                    rb, [jnp.full((16,), par, jnp.int32),
                         g16 + iota16,
                         jnp.full((16,), C - 15, jnp.int32)])
                e = ss + plsc.load_gather(sdst_t, [d16])
                e = jnp.where(e >= 0, e, e * jnp.float32(0.2))
                pb[pl.ds(g16, 16)] = jnp.exp(e)
                return 0
            lax.fori_loop(0, BB // 16, _pw, 0)

            def _scale(ei, _):
                pe = jnp.full((16,), pb[pl.ds(ei, 16)][0], jnp.float32)
                for j in range(nj):
                    rb[par, ei, pl.ds(j * 16, 16)] = (
                        rb[par, ei, pl.ds(j * 16, 16)] * pe)
                return 0
            lax.fori_loop(0, BB, _scale, 0)

            pltpu.async_copy(rb.at[par], acc_sh.at[cmb.at[q, 1]],
                             ssems[par], add=True)

            @pl.when(b + 2 < NB)
            def _():
                @pl.when(b >= 1)
                def _():
                    pltpu.make_async_copy(rb.at[nxt],
                                          acc_sh.at[cmb.at[0, 1]],
                                          ssems[nxt]).wait()
                pltpu.make_async_copy(srcdst.at[h, wid, 0], cmb.at[qn],
                                      isems[qn]).wait()
                pltpu.async_copy(tab.at[cmb.at[qn, 0]], rb.at[nxt],
                                 gsems[nxt])

            @pl.when(b + 5 < NB)
            def _():
                pltpu.async_copy(srcdst.at[h, wid, b + 5], cmb.at[qf],
                                 isems[qf])

        def _hex(t, _):
            for u in range(NI):
                _batch(t * NI + u, u % NBUF, u)
            return 0
        lax.fori_loop(0, NB // NI, _hex, 0)
        for par in range(NBUF):
            pltpu.make_async_copy(rb.at[par], acc_sh.at[cmb.at[0, 1]],
                                  ssems[par]).wait()
        plsc.subcore_barrier()
        pltpu.sync_copy(acc_sh.at[pl.ds(s * RPT, RPT)],
                        acc_out.at[c, h, pl.ds(s * RPT, RPT)])
        plsc.subcore_barrier()


def _edge(tab, srcdst, sdst, heads, C, BB, NB):
    mesh = plsc.VectorSubcoreMesh(core_axis_name="c", subcore_axis_name="s")
    return pl.kernel(
        functools.partial(_edge_body, heads, C, BB, NB),
        out_type=jax.ShapeDtypeStruct((2, heads, R, C), jnp.float32),
        mesh=mesh,
        compiler_params=pltpu.CompilerParams(needs_layout_passes=False,
                                             use_tc_tiling_on_sc=False),
        scratch_types=[
            pltpu.VMEM((NBUF, BB, C), jnp.float32),
            pltpu.VMEM((NI, 2, BB), jnp.int32),
            pltpu.VMEM((R,), jnp.float32),
            pltpu.VMEM((BB + 16,), jnp.float32),
            pltpu.VMEM_SHARED((R, C), jnp.float32),
        ] + [pltpu.SemaphoreType.DMA] * (2 * NBUF + NI),
    )(tab, srcdst, sdst)


def kernel(x, edge_index, batch, W_e0, a_src_e0, a_dst_e0, b_e0, W_e1,
           a_src_e1, a_dst_e1, b_e1, Wg1, bg1, Wg2, bg2, W_d0, a_src_d0,
           a_dst_d0, b_d0, W_d1, a_src_d1, a_dst_d1, b_d1):
    f32 = jnp.float32
    i32 = jnp.int32

    x_pad = jnp.zeros((R, 128), f32).at[:N].set(x)
    loops = jnp.arange(N, dtype=i32)
    src0 = jnp.concatenate([edge_index[0].astype(i32), loops])
    dst0 = jnp.concatenate([edge_index[1].astype(i32), loops])
    batch2d = jnp.full((R,), 16, i32).at[:N].set(batch.astype(i32)).reshape(
        R, 1)

    def mk_srcdst(heads, BB, NB):
        EPL = NW * NB * BB
        srcp = jnp.full((EPL,), N, i32).at[:ET].set(src0)
        dstp = jnp.full((EPL,), N, i32).at[:ET].set(dst0)
        dst5 = jnp.broadcast_to(dstp.reshape(1, NW, NB, 1, BB),
                                (heads, NW, NB, 1, BB))
        srcoff = (srcp[None, :]
                  + (jnp.arange(heads, dtype=i32) * R)[:, None]
                  ).reshape(heads, NW, NB, 1, BB)
        return jnp.concatenate([srcoff, dst5], axis=3)

    sd_e0 = mk_srcdst(8, 64, 162)
    sd_e1 = mk_srcdst(8, 64, 162)
    sd_d = mk_srcdst(1, 64, 162)

    tab, sdst = _prep(x_pad, W_e0, a_src_e0.reshape(8, 128),
                      a_dst_e0.reshape(8, 128), 8, 128)
    acc = _edge(tab.reshape(8 * R, 144), sd_e0, sdst, 8, 144, 64, 162)
    h = _fin(acc, b_e0, 8, 128, relu=True)

    tab, sdst = _prep(h, W_e1, a_src_e1.reshape(8, 64),
                      a_dst_e1.reshape(8, 64), 8, 64)
    acc = _edge(tab.reshape(8 * R, 80), sd_e1, sdst, 8, 80, 64, 162)
    h = _fin(acc, b_e1, 8, 64, relu=False)

    h = _pool(h, batch2d, Wg1, bg1, Wg2, bg2.reshape(1, 1))

    tab, sdst = _prep(h, W_d0, a_src_d0.reshape(1, 128),
                      a_dst_d0.reshape(1, 128), 1, 128)
    acc = _edge(tab.reshape(R, 144), sd_d, sdst, 1, 144, 64, 162)
    h = _fin(acc, b_d0, 1, 128, relu=True)

    tab, sdst = _prep(h, W_d1, a_src_d1.reshape(1, 128),
                      a_dst_d1.reshape(1, 128), 1, 128)
    acc = _edge(tab.reshape(R, 144), sd_d, sdst, 1, 144, 64, 162)
    h = _fin(acc, b_d1, 1, 128, relu=False)

    return h[:N]


# speedup vs baseline: 1.4086x; 1.0111x over previous
import functools

import jax
import jax.numpy as jnp
from jax import lax
from jax.experimental import pallas as pl
from jax.experimental.pallas import tpu as pltpu
from jax.experimental.pallas import tpu_sc as plsc

N = 10000
E = 320000
ET = E + N
R = 10240
NW = 32
NBUF = 3
NI = 6
RPT = R // 16
BR = 256


def _prep_body(heads, oc, x_ref, w_ref, asr_ref, adr_ref, tab_ref, sdst_ref):
    xb = x_ref[...]
    h = jnp.dot(xb, w_ref[...], preferred_element_type=jnp.float32)
    ones = jnp.ones((BR, 1), jnp.float32)
    zpad = jnp.zeros((BR, 14), jnp.float32)
    for k in range(heads):
        hk = h[:, k * oc:(k + 1) * oc]
        ssrc = jnp.sum(hk * asr_ref[k][None, :], axis=1, keepdims=True)
        tab_ref[k] = jnp.concatenate([hk, ones, ssrc, zpad], axis=1)
        sdst_ref[k] = jnp.sum(hk * adr_ref[k][None, :], axis=1)


def _prep(x, W, a_src, a_dst, heads, oc):
    ind = x.shape[1]
    C = oc + 16
    grid = R // BR
    return pl.pallas_call(
        functools.partial(_prep_body, heads, oc),
        grid=(grid,),
        in_specs=[
            pl.BlockSpec((BR, ind), lambda i: (i, 0)),
            pl.BlockSpec((ind, heads * oc), lambda i: (0, 0)),
            pl.BlockSpec((heads, oc), lambda i: (0, 0)),
            pl.BlockSpec((heads, oc), lambda i: (0, 0)),
        ],
        out_specs=[
            pl.BlockSpec((heads, BR, C), lambda i: (0, i, 0)),
            pl.BlockSpec((heads, BR), lambda i: (0, i)),
        ],
        out_shape=[
            jax.ShapeDtypeStruct((heads, R, C), jnp.float32),
            jax.ShapeDtypeStruct((heads, R), jnp.float32),
        ],
    )(x, W, a_src, a_dst)


def _fin_body(heads, oc, relu, bias_ref, acc_ref, o_ref):
    acc = acc_ref[...]
    num = acc[0, :, :, :oc] + acc[1, :, :, :oc]
    z = acc[0, :, :, oc:oc + 1] + acc[1, :, :, oc:oc + 1]
    o = jnp.sum(num / (z + 1e-16), axis=0) * (1.0 / heads) + bias_ref[...][None, :]
    if relu:
        o = jnp.maximum(o, 0.0)
    o_ref[...] = o


def _fin(acc, bias, heads, oc, relu):
    C = oc + 16
    grid = R // BR
    return pl.pallas_call(
        functools.partial(_fin_body, heads, oc, relu),
        grid=(grid,),
        in_specs=[
            pl.BlockSpec((oc,), lambda i: (0,)),
            pl.BlockSpec((2, heads, BR, C), lambda i: (0, 0, i, 0)),
        ],
        out_specs=pl.BlockSpec((BR, oc), lambda i: (i, 0)),
        out_shape=jax.ShapeDtypeStruct((R, oc), jnp.float32),
    )(bias, acc)


def _finprep_body(hi, oci, relu, ho, oco, bias_ref, acc_ref, w_ref, asr_ref,
                  adr_ref, tab_ref, sdst_ref):
    acc = acc_ref[...]
    num = acc[0, :, :, :oci] + acc[1, :, :, :oci]
    z = acc[0, :, :, oci:oci + 1] + acc[1, :, :, oci:oci + 1]
    o = jnp.sum(num / (z + 1e-16), axis=0) * (1.0 / hi) + bias_ref[...][None, :]
    if relu:
        o = jnp.maximum(o, 0.0)
    h = jnp.dot(o, w_ref[...], preferred_element_type=jnp.float32)
    ones = jnp.ones((BR, 1), jnp.float32)
    zpad = jnp.zeros((BR, 14), jnp.float32)
    for k in range(ho):
        hk = h[:, k * oco:(k + 1) * oco]
        ssrc = jnp.sum(hk * asr_ref[k][None, :], axis=1, keepdims=True)
        tab_ref[k] = jnp.concatenate([hk, ones, ssrc, zpad], axis=1)
        sdst_ref[k] = jnp.sum(hk * adr_ref[k][None, :], axis=1)


def _finprep(acc, bias, W, a_src, a_dst, hi, oci, relu, ho, oco):
    Ci, Co = oci + 16, oco + 16
    grid = R // BR
    return pl.pallas_call(
        functools.partial(_finprep_body, hi, oci, relu, ho, oco),
        grid=(grid,),
        in_specs=[
            pl.BlockSpec((oci,), lambda i: (0,)),
            pl.BlockSpec((2, hi, BR, Ci), lambda i: (0, 0, i, 0)),
            pl.BlockSpec((oci, ho * oco), lambda i: (0, 0)),
            pl.BlockSpec((ho, oco), lambda i: (0, 0)),
            pl.BlockSpec((ho, oco), lambda i: (0, 0)),
        ],
        out_specs=[
            pl.BlockSpec((ho, BR, Co), lambda i: (0, i, 0)),
            pl.BlockSpec((ho, BR), lambda i: (0, i)),
        ],
        out_shape=[
            jax.ShapeDtypeStruct((ho, R, Co), jnp.float32),
            jax.ShapeDtypeStruct((ho, R), jnp.float32),
        ],
    )(bias, acc, W, a_src, a_dst)


def _poolprep_body(h_ref, b_ref, wg1_ref, bg1_ref, wg2_ref, bg2_ref, w_ref,
                   asr_ref, adr_ref, tab_ref, sdst_ref):
    hh = h_ref[...]
    t = jnp.maximum(
        jnp.dot(hh, wg1_ref[...], preferred_element_type=jnp.float32)
        + bg1_ref[...][None, :], 0.0)
    g = jnp.dot(t, wg2_ref[...], preferred_element_type=jnp.float32) \
        + bg2_ref[...]
    oh = (b_ref[...] == lax.broadcasted_iota(jnp.int32, (R, 16), 1)).astype(
        jnp.float32)
    m = jnp.max(jnp.where(oh > 0, g, -3e38), axis=0, keepdims=True)
    mn = jnp.sum(oh * m, axis=1, keepdims=True)
    p = jnp.exp(g - mn)
    z = jnp.sum(oh * p, axis=0, keepdims=True)
    zn = jnp.sum(oh * z, axis=1, keepdims=True)
    w = oh * (p / (zn + 1e-16))
    pooled = lax.dot_general(w, hh, (((0,), (0,)), ((), ())),
                             preferred_element_type=jnp.float32)
    x = jnp.dot(oh, pooled, preferred_element_type=jnp.float32)
    h = jnp.dot(x, w_ref[...], preferred_element_type=jnp.float32)
    ssrc = jnp.sum(h * asr_ref[0][None, :], axis=1, keepdims=True)
    ones = jnp.ones((R, 1), jnp.float32)
    zpad = jnp.zeros((R, 14), jnp.float32)
    tab_ref[0] = jnp.concatenate([h, ones, ssrc, zpad], axis=1)
    sdst_ref[0] = jnp.sum(h * adr_ref[0][None, :], axis=1)


def _poolprep(h, batch2d, Wg1, bg1, Wg2, bg2, W, a_src, a_dst, oco):
    Co = oco + 16
    return pl.pallas_call(
        _poolprep_body,
        out_shape=[
            jax.ShapeDtypeStruct((1, R, Co), jnp.float32),
            jax.ShapeDtypeStruct((1, R), jnp.float32),
        ],
    )(h, batch2d, Wg1, bg1, Wg2, bg2, W, a_src, a_dst)


def _edge_body(heads, C, BB, NB, tab, srcdst, sdst, acc_out, rb, cmb, sdst_t,
               pb, acc_sh, *sems):
    gsems = sems[0:NBUF]
    ssems = sems[NBUF:2 * NBUF]
    isems = sems[2 * NBUF:2 * NBUF + NI]
    c = lax.axis_index("c")
    s = lax.axis_index("s")
    wid = s * 2 + c
    nj = C // 16
    iota16 = lax.broadcasted_iota(jnp.int32, (16,), 0)

    for h in range(heads):
        pltpu.sync_copy(sdst.at[h], sdst_t)

        def _zb(ei, _):
            for j in range(nj):
                rb[0, ei, pl.ds(j * 16, 16)] = jnp.zeros((16,), jnp.float32)
            return 0
        lax.fori_loop(0, BB, _zb, 0)
        for k in range(RPT // BB):
            pltpu.sync_copy(rb.at[0],
                            acc_sh.at[pl.ds(s * RPT + k * BB, BB)])
        rem = RPT - (RPT // BB) * BB
        if rem:
            pltpu.sync_copy(
                rb.at[0, pl.ds(0, rem)],
                acc_sh.at[pl.ds(s * RPT + (RPT // BB) * BB, rem)])
        plsc.subcore_barrier()

        for b0 in range(5):
            pltpu.async_copy(srcdst.at[h, wid, b0], cmb.at[b0], isems[b0])
        for b0 in range(2):
            pltpu.make_async_copy(srcdst.at[h, wid, b0], cmb.at[b0],
                                  isems[b0]).wait()
            pltpu.async_copy(tab.at[cmb.at[b0, 0]], rb.at[b0], gsems[b0])

        def _batch(b, par, q):
            nxt = (par + 2) % NBUF
            qn = (q + 2) % NI
            qf = (q + 5) % NI
            pltpu.make_async_copy(tab.at[cmb.at[0, 0]], rb.at[par],
                                  gsems[par]).wait()
            def _pw(g, _):
                g16 = g * 16
                d16 = cmb[q, 1, pl.ds(g16, 16)]
                ss = plsc.load_gather(
                    rb, [jnp.full((16,), par, jnp.int32),
                         g16 + iota16,
                         jnp.full((16,), C - 15, jnp.int32)])
                e = ss + plsc.load_gather(sdst_t, [d16])
                e = jnp.where(e >= 0, e, e * jnp.float32(0.2))
                pb[pl.ds(g16, 16)] = jnp.exp(e)
                return 0
            lax.fori_loop(0, BB // 16, _pw, 0)

            def _scale(ei, _):
                pe = jnp.full((16,), pb[pl.ds(ei, 16)][0], jnp.float32)
                for j in range(nj):
                    rb[par, ei, pl.ds(j * 16, 16)] = (
                        rb[par, ei, pl.ds(j * 16, 16)] * pe)
                return 0
            lax.fori_loop(0, BB, _scale, 0)

            pltpu.async_copy(rb.at[par], acc_sh.at[cmb.at[q, 1]],
                             ssems[par], add=True)

            @pl.when(b + 2 < NB)
            def _():
                @pl.when(b >= 1)
                def _():
                    pltpu.make_async_copy(rb.at[nxt],
                                          acc_sh.at[cmb.at[0, 1]],
                                          ssems[nxt]).wait()
                pltpu.make_async_copy(srcdst.at[h, wid, 0], cmb.at[qn],
                                      isems[qn]).wait()
                pltpu.async_copy(tab.at[cmb.at[qn, 0]], rb.at[nxt],
                                 gsems[nxt])

            @pl.when(b + 5 < NB)
            def _():
                pltpu.async_copy(srcdst.at[h, wid, b + 5], cmb.at[qf],
                                 isems[qf])

        def _hex(t, _):
            for u in range(NI):
                _batch(t * NI + u, u % NBUF, u)
            return 0
        lax.fori_loop(0, NB // NI, _hex, 0)
        for par in range(NBUF):
            pltpu.make_async_copy(rb.at[par], acc_sh.at[cmb.at[0, 1]],
                                  ssems[par]).wait()
        plsc.subcore_barrier()
        pltpu.sync_copy(acc_sh.at[pl.ds(s * RPT, RPT)],
                        acc_out.at[c, h, pl.ds(s * RPT, RPT)])
        plsc.subcore_barrier()


def _edge(tab, srcdst, sdst, heads, C, BB, NB):
    mesh = plsc.VectorSubcoreMesh(core_axis_name="c", subcore_axis_name="s")
    return pl.kernel(
        functools.partial(_edge_body, heads, C, BB, NB),
        out_type=jax.ShapeDtypeStruct((2, heads, R, C), jnp.float32),
        mesh=mesh,
        compiler_params=pltpu.CompilerParams(needs_layout_passes=False,
                                             use_tc_tiling_on_sc=False),
        scratch_types=[
            pltpu.VMEM((NBUF, BB, C), jnp.float32),
            pltpu.VMEM((NI, 2, BB), jnp.int32),
            pltpu.VMEM((R,), jnp.float32),
            pltpu.VMEM((BB + 16,), jnp.float32),
            pltpu.VMEM_SHARED((R, C), jnp.float32),
        ] + [pltpu.SemaphoreType.DMA] * (2 * NBUF + NI),
    )(tab, srcdst, sdst)


def kernel(x, edge_index, batch, W_e0, a_src_e0, a_dst_e0, b_e0, W_e1,
           a_src_e1, a_dst_e1, b_e1, Wg1, bg1, Wg2, bg2, W_d0, a_src_d0,
           a_dst_d0, b_d0, W_d1, a_src_d1, a_dst_d1, b_d1):
    f32 = jnp.float32
    i32 = jnp.int32

    x_pad = jnp.zeros((R, 128), f32).at[:N].set(x)
    loops = jnp.arange(N, dtype=i32)
    src0 = jnp.concatenate([edge_index[0].astype(i32), loops])
    dst0 = jnp.concatenate([edge_index[1].astype(i32), loops])
    batch2d = jnp.full((R,), 16, i32).at[:N].set(batch.astype(i32)).reshape(
        R, 1)

    def mk_srcdst(heads, BB, NB):
        EPL = NW * NB * BB
        srcp = jnp.full((EPL,), N, i32).at[:ET].set(src0)
        dstp = jnp.full((EPL,), N, i32).at[:ET].set(dst0)
        dst5 = jnp.broadcast_to(dstp.reshape(1, NW, NB, 1, BB),
                                (heads, NW, NB, 1, BB))
        srcoff = (srcp[None, :]
                  + (jnp.arange(heads, dtype=i32) * R)[:, None]
                  ).reshape(heads, NW, NB, 1, BB)
        return jnp.concatenate([srcoff, dst5], axis=3)

    sd8 = mk_srcdst(8, 64, 162)
    sd1 = sd8[:1]

    tab, sdst = _prep(x_pad, W_e0, a_src_e0.reshape(8, 128),
                      a_dst_e0.reshape(8, 128), 8, 128)
    acc = _edge(tab.reshape(8 * R, 144), sd8, sdst, 8, 144, 64, 162)

    tab, sdst = _finprep(acc, b_e0, W_e1, a_src_e1.reshape(8, 64),
                         a_dst_e1.reshape(8, 64), 8, 128, True, 8, 64)
    acc = _edge(tab.reshape(8 * R, 80), sd8, sdst, 8, 80, 64, 162)
    h = _fin(acc, b_e1, 8, 64, relu=False)

    tab, sdst = _poolprep(h, batch2d, Wg1, bg1, Wg2, bg2.reshape(1, 1),
                          W_d0, a_src_d0.reshape(1, 128),
                          a_dst_d0.reshape(1, 128), 128)
    acc = _edge(tab.reshape(R, 144), sd1, sdst, 1, 144, 64, 162)

    tab, sdst = _finprep(acc, b_d0, W_d1, a_src_d1.reshape(1, 128),
                         a_dst_d1.reshape(1, 128), 1, 128, True, 1, 128)
    acc = _edge(tab.reshape(R, 144), sd1, sdst, 1, 144, 64, 162)
    h = _fin(acc, b_d1, 1, 128, relu=False)

    return h[:N]


# speedup vs baseline: 1.4245x; 1.0113x over previous
import functools

import jax
import jax.numpy as jnp
from jax import lax
from jax.experimental import pallas as pl
from jax.experimental.pallas import tpu as pltpu
from jax.experimental.pallas import tpu_sc as plsc

N = 10000
E = 320000
ET = E + N
R = 10240
NW = 32
NBUF = 3
NI = 6
RPT = R // 16
BR = 256


def _prep_body(heads, oc, x_ref, w_ref, asr_ref, adr_ref, tab_ref, sdst_ref):
    xb = x_ref[...]
    h = jnp.dot(xb, w_ref[...], preferred_element_type=jnp.float32)
    ones = jnp.ones((BR, 1), jnp.float32)
    zpad = jnp.zeros((BR, 14), jnp.float32)
    for k in range(heads):
        hk = h[:, k * oc:(k + 1) * oc]
        ssrc = jnp.sum(hk * asr_ref[k][None, :], axis=1, keepdims=True)
        tab_ref[k] = jnp.concatenate([hk, ones, ssrc, zpad], axis=1)
        sdst_ref[k] = jnp.sum(hk * adr_ref[k][None, :], axis=1)


def _prep(x, W, a_src, a_dst, heads, oc):
    ind = x.shape[1]
    C = oc + 16
    grid = R // BR
    return pl.pallas_call(
        functools.partial(_prep_body, heads, oc),
        grid=(grid,),
        in_specs=[
            pl.BlockSpec((BR, ind), lambda i: (i, 0)),
            pl.BlockSpec((ind, heads * oc), lambda i: (0, 0)),
            pl.BlockSpec((heads, oc), lambda i: (0, 0)),
            pl.BlockSpec((heads, oc), lambda i: (0, 0)),
        ],
        out_specs=[
            pl.BlockSpec((heads, BR, C), lambda i: (0, i, 0)),
            pl.BlockSpec((heads, BR), lambda i: (0, i)),
        ],
        out_shape=[
            jax.ShapeDtypeStruct((heads, R, C), jnp.float32),
            jax.ShapeDtypeStruct((heads, R), jnp.float32),
        ],
    )(x, W, a_src, a_dst)


def _fin_body(heads, oc, relu, bias_ref, acc_ref, o_ref):
    acc = acc_ref[...]
    num = acc[0, :, :, :oc] + acc[1, :, :, :oc]
    z = acc[0, :, :, oc:oc + 1] + acc[1, :, :, oc:oc + 1]
    o = jnp.sum(num / (z + 1e-16), axis=0) * (1.0 / heads) + bias_ref[...][None, :]
    if relu:
        o = jnp.maximum(o, 0.0)
    o_ref[...] = o


def _fin(acc, bias, heads, oc, relu):
    C = oc + 16
    grid = R // BR
    return pl.pallas_call(
        functools.partial(_fin_body, heads, oc, relu),
        grid=(grid,),
        in_specs=[
            pl.BlockSpec((oc,), lambda i: (0,)),
            pl.BlockSpec((2, heads, BR, C), lambda i: (0, 0, i, 0)),
        ],
        out_specs=pl.BlockSpec((BR, oc), lambda i: (i, 0)),
        out_shape=jax.ShapeDtypeStruct((R, oc), jnp.float32),
    )(bias, acc)


def _finprep_body(hi, oci, relu, ho, oco, bias_ref, acc_ref, w_ref, asr_ref,
                  adr_ref, tab_ref, sdst_ref):
    acc = acc_ref[...]
    num = acc[0, :, :, :oci] + acc[1, :, :, :oci]
    z = acc[0, :, :, oci:oci + 1] + acc[1, :, :, oci:oci + 1]
    o = jnp.sum(num / (z + 1e-16), axis=0) * (1.0 / hi) + bias_ref[...][None, :]
    if relu:
        o = jnp.maximum(o, 0.0)
    h = jnp.dot(o, w_ref[...], preferred_element_type=jnp.float32)
    ones = jnp.ones((BR, 1), jnp.float32)
    zpad = jnp.zeros((BR, 14), jnp.float32)
    for k in range(ho):
        hk = h[:, k * oco:(k + 1) * oco]
        ssrc = jnp.sum(hk * asr_ref[k][None, :], axis=1, keepdims=True)
        tab_ref[k] = jnp.concatenate([hk, ones, ssrc, zpad], axis=1)
        sdst_ref[k] = jnp.sum(hk * adr_ref[k][None, :], axis=1)


def _finprep(acc, bias, W, a_src, a_dst, hi, oci, relu, ho, oco):
    Ci, Co = oci + 16, oco + 16
    grid = R // BR
    return pl.pallas_call(
        functools.partial(_finprep_body, hi, oci, relu, ho, oco),
        grid=(grid,),
        in_specs=[
            pl.BlockSpec((oci,), lambda i: (0,)),
            pl.BlockSpec((2, hi, BR, Ci), lambda i: (0, 0, i, 0)),
            pl.BlockSpec((oci, ho * oco), lambda i: (0, 0)),
            pl.BlockSpec((ho, oco), lambda i: (0, 0)),
            pl.BlockSpec((ho, oco), lambda i: (0, 0)),
        ],
        out_specs=[
            pl.BlockSpec((ho, BR, Co), lambda i: (0, i, 0)),
            pl.BlockSpec((ho, BR), lambda i: (0, i)),
        ],
        out_shape=[
            jax.ShapeDtypeStruct((ho, R, Co), jnp.float32),
            jax.ShapeDtypeStruct((ho, R), jnp.float32),
        ],
    )(bias, acc, W, a_src, a_dst)


def _poolprep_body(h_ref, b_ref, wg1_ref, bg1_ref, wg2_ref, bg2_ref, w_ref,
                   asr_ref, adr_ref, tab_ref, sdst_ref):
    hh = h_ref[...]
    t = jnp.maximum(
        jnp.dot(hh, wg1_ref[...], preferred_element_type=jnp.float32)
        + bg1_ref[...][None, :], 0.0)
    g = jnp.dot(t, wg2_ref[...], preferred_element_type=jnp.float32) \
        + bg2_ref[...]
    oh = (b_ref[...] == lax.broadcasted_iota(jnp.int32, (R, 16), 1)).astype(
        jnp.float32)
    m = jnp.max(jnp.where(oh > 0, g, -3e38), axis=0, keepdims=True)
    mn = jnp.sum(oh * m, axis=1, keepdims=True)
    p = jnp.exp(g - mn)
    z = jnp.sum(oh * p, axis=0, keepdims=True)
    zn = jnp.sum(oh * z, axis=1, keepdims=True)
    w = oh * (p / (zn + 1e-16))
    pooled = lax.dot_general(w, hh, (((0,), (0,)), ((), ())),
                             preferred_element_type=jnp.float32)
    x = jnp.dot(oh, pooled, preferred_element_type=jnp.float32)
    h = jnp.dot(x, w_ref[...], preferred_element_type=jnp.float32)
    ssrc = jnp.sum(h * asr_ref[0][None, :], axis=1, keepdims=True)
    ones = jnp.ones((R, 1), jnp.float32)
    zpad = jnp.zeros((R, 14), jnp.float32)
    tab_ref[0] = jnp.concatenate([h, ones, ssrc, zpad], axis=1)
    sdst_ref[0] = jnp.sum(h * adr_ref[0][None, :], axis=1)


def _poolprep(h, batch2d, Wg1, bg1, Wg2, bg2, W, a_src, a_dst, oco):
    Co = oco + 16
    return pl.pallas_call(
        _poolprep_body,
        out_shape=[
            jax.ShapeDtypeStruct((1, R, Co), jnp.float32),
            jax.ShapeDtypeStruct((1, R), jnp.float32),
        ],
    )(h, batch2d, Wg1, bg1, Wg2, bg2, W, a_src, a_dst)


def _edge_body(heads, C, BB, NB, tab, srcdst, sdst, acc_out, rb, cmb, sdst_t,
               pb, acc_sh, *sems):
    gsems = sems[0:NBUF]
    ssems = sems[NBUF:2 * NBUF]
    isems = sems[2 * NBUF:2 * NBUF + NI]
    zsem = sems[2 * NBUF + NI]
    dsem = sems[2 * NBUF + NI + 1]
    c = lax.axis_index("c")
    s = lax.axis_index("s")
    wid = s * 2 + c
    nj = C // 16
    iota16 = lax.broadcasted_iota(jnp.int32, (16,), 0)

    for h in range(heads):
        pltpu.async_copy(sdst.at[h], sdst_t, dsem)

        def _zb(ei, _):
            for j in range(nj):
                rb[0, ei, pl.ds(j * 16, 16)] = jnp.zeros((16,), jnp.float32)
            return 0
        lax.fori_loop(0, BB, _zb, 0)
        nzc = RPT // BB
        rem = RPT - nzc * BB
        for k in range(nzc):
            pltpu.async_copy(rb.at[0],
                             acc_sh.at[pl.ds(s * RPT + k * BB, BB)], zsem)
        if rem:
            pltpu.async_copy(
                rb.at[0, pl.ds(0, rem)],
                acc_sh.at[pl.ds(s * RPT + nzc * BB, rem)], zsem)
        for k in range(nzc):
            pltpu.make_async_copy(rb.at[0], acc_sh.at[pl.ds(0, BB)],
                                  zsem).wait()
        if rem:
            pltpu.make_async_copy(rb.at[0, pl.ds(0, rem)],
                                  acc_sh.at[pl.ds(0, rem)], zsem).wait()
        plsc.subcore_barrier()

        for b0 in range(5):
            pltpu.async_copy(srcdst.at[h, wid, b0], cmb.at[b0], isems[b0])
        for b0 in range(2):
            pltpu.make_async_copy(srcdst.at[h, wid, b0], cmb.at[b0],
                                  isems[b0]).wait()
            pltpu.async_copy(tab.at[cmb.at[b0, 0]], rb.at[b0], gsems[b0])
        pltpu.make_async_copy(sdst.at[h], sdst_t, dsem).wait()

        def _batch(b, par, q):
            nxt = (par + 2) % NBUF
            qn = (q + 2) % NI
            qf = (q + 5) % NI
            pltpu.make_async_copy(tab.at[cmb.at[0, 0]], rb.at[par],
                                  gsems[par]).wait()
            def _pw(g, _):
                g16 = g * 16
                d16 = cmb[q, 1, pl.ds(g16, 16)]
                ss = plsc.load_gather(
                    rb, [jnp.full((16,), par, jnp.int32),
                         g16 + iota16,
                         jnp.full((16,), C - 15, jnp.int32)])
                e = ss + plsc.load_gather(sdst_t, [d16])
                e = jnp.where(e >= 0, e, e * jnp.float32(0.2))
                pb[pl.ds(g16, 16)] = jnp.exp(e)
                return 0
            lax.fori_loop(0, BB // 16, _pw, 0)

            def _scale(ei, _):
                pe = jnp.full((16,), pb[pl.ds(ei, 16)][0], jnp.float32)
                for j in range(nj):
                    rb[par, ei, pl.ds(j * 16, 16)] = (
                        rb[par, ei, pl.ds(j * 16, 16)] * pe)
                return 0
            lax.fori_loop(0, BB, _scale, 0)

            pltpu.async_copy(rb.at[par], acc_sh.at[cmb.at[q, 1]],
                             ssems[par], add=True)

            @pl.when(b + 2 < NB)
            def _():
                @pl.when(b >= 1)
                def _():
                    pltpu.make_async_copy(rb.at[nxt],
                                          acc_sh.at[cmb.at[0, 1]],
                                          ssems[nxt]).wait()
                pltpu.make_async_copy(srcdst.at[h, wid, 0], cmb.at[qn],
                                      isems[qn]).wait()
                pltpu.async_copy(tab.at[cmb.at[qn, 0]], rb.at[nxt],
                                 gsems[nxt])

            @pl.when(b + 5 < NB)
            def _():
                pltpu.async_copy(srcdst.at[h, wid, b + 5], cmb.at[qf],
                                 isems[qf])

        def _hex(t, _):
            for u in range(NI):
                _batch(t * NI + u, u % NBUF, u)
            return 0
        lax.fori_loop(0, NB // NI, _hex, 0)
        for par in range(NBUF):
            pltpu.make_async_copy(rb.at[par], acc_sh.at[cmb.at[0, 1]],
                                  ssems[par]).wait()
        plsc.subcore_barrier()
        pltpu.sync_copy(acc_sh.at[pl.ds(s * RPT, RPT)],
                        acc_out.at[c, h, pl.ds(s * RPT, RPT)])


def _edge(tab, srcdst, sdst, heads, C, BB, NB):
    mesh = plsc.VectorSubcoreMesh(core_axis_name="c", subcore_axis_name="s")
    return pl.kernel(
        functools.partial(_edge_body, heads, C, BB, NB),
        out_type=jax.ShapeDtypeStruct((2, heads, R, C), jnp.float32),
        mesh=mesh,
        compiler_params=pltpu.CompilerParams(needs_layout_passes=False,
                                             use_tc_tiling_on_sc=False),
        scratch_types=[
            pltpu.VMEM((NBUF, BB, C), jnp.float32),
            pltpu.VMEM((NI, 2, BB), jnp.int32),
            pltpu.VMEM((R,), jnp.float32),
            pltpu.VMEM((BB + 16,), jnp.float32),
            pltpu.VMEM_SHARED((R, C), jnp.float32),
        ] + [pltpu.SemaphoreType.DMA] * (2 * NBUF + NI + 2),
    )(tab, srcdst, sdst)


def kernel(x, edge_index, batch, W_e0, a_src_e0, a_dst_e0, b_e0, W_e1,
           a_src_e1, a_dst_e1, b_e1, Wg1, bg1, Wg2, bg2, W_d0, a_src_d0,
           a_dst_d0, b_d0, W_d1, a_src_d1, a_dst_d1, b_d1):
    f32 = jnp.float32
    i32 = jnp.int32

    x_pad = jnp.zeros((R, 128), f32).at[:N].set(x)
    loops = jnp.arange(N, dtype=i32)
    src0 = jnp.concatenate([edge_index[0].astype(i32), loops])
    dst0 = jnp.concatenate([edge_index[1].astype(i32), loops])
    batch2d = jnp.full((R,), 16, i32).at[:N].set(batch.astype(i32)).reshape(
        R, 1)

    def mk_srcdst(heads, BB, NB):
        EPL = NW * NB * BB
        srcp = jnp.full((EPL,), N, i32).at[:ET].set(src0)
        dstp = jnp.full((EPL,), N, i32).at[:ET].set(dst0)
        dst5 = jnp.broadcast_to(dstp.reshape(1, NW, NB, 1, BB),
                                (heads, NW, NB, 1, BB))
        srcoff = (srcp[None, :]
                  + (jnp.arange(heads, dtype=i32) * R)[:, None]
                  ).reshape(heads, NW, NB, 1, BB)
        return jnp.concatenate([srcoff, dst5], axis=3)

    sd8 = mk_srcdst(8, 64, 162)
    sd1 = sd8[:1]

    tab, sdst = _prep(x_pad, W_e0, a_src_e0.reshape(8, 128),
                      a_dst_e0.reshape(8, 128), 8, 128)
    acc = _edge(tab.reshape(8 * R, 144), sd8, sdst, 8, 144, 64, 162)

    tab, sdst = _finprep(acc, b_e0, W_e1, a_src_e1.reshape(8, 64),
                         a_dst_e1.reshape(8, 64), 8, 128, True, 8, 64)
    acc = _edge(tab.reshape(8 * R, 80), sd8, sdst, 8, 80, 64, 162)
    h = _fin(acc, b_e1, 8, 64, relu=False)

    tab, sdst = _poolprep(h, batch2d, Wg1, bg1, Wg2, bg2.reshape(1, 1),
                          W_d0, a_src_d0.reshape(1, 128),
                          a_dst_d0.reshape(1, 128), 128)
    acc = _edge(tab.reshape(R, 144), sd1, sdst, 1, 144, 64, 162)

    tab, sdst = _finprep(acc, b_d0, W_d1, a_src_d1.reshape(1, 128),
                         a_dst_d1.reshape(1, 128), 1, 128, True, 1, 128)
    acc = _edge(tab.reshape(R, 144), sd1, sdst, 1, 144, 64, 162)
    h = _fin(acc, b_d1, 1, 128, relu=False)

    return h[:N]
